# baseline (speedup 1.0000x reference)
import math

import jax
import jax.numpy as jnp
import numpy as np
from jax import lax
from jax.experimental import pallas as pl
from jax.experimental.pallas import tpu as pltpu
from jax.experimental.pallas import tpu_sc as plsc

_VOCAB = 1000000
_D = 64
_MAX_LEN = 512
_BATCH = 4096
_SEQ = 200

_NC = 2
_NS = 16
_NW = _NC * _NS

_TOTAL_ROWS = _BATCH * _SEQ
_ROWS_PER_W = _TOTAL_ROWS // _NW
_CHUNK = 400
_IDXW = 80
_GSUB = _CHUNK // _IDXW
_CHUNKS_PER_W = _ROWS_PER_W // _CHUNK
_LANES = 16
_VPR = _D // _LANES


def _build_pe_np(max_len, d_model):
    pe = np.zeros((max_len, d_model), dtype=np.float32)
    position = np.arange(0, max_len, dtype=np.float32)[:, None]
    div_term = np.exp(
        np.arange(0, d_model, 2, dtype=np.float32) * (-math.log(10000.0) / d_model)
    )
    pe[:, 0::2] = np.sin(position * div_term)
    pe[:, 1::2] = np.cos(position * div_term)
    return pe


_PE2 = np.tile(_build_pe_np(_MAX_LEN, _D)[:_SEQ], (2, 1))


def _sc_body(ids_hbm, pe_hbm, table_hbm, out_hbm,
             pe_v, idx_v, rows_v, sem_g, sem_o):
    wid = lax.axis_index("s") * _NC + lax.axis_index("c")
    row_base = wid * _ROWS_PER_W

    pltpu.sync_copy(pe_hbm, pe_v)

    def chunk_body(c, carry):
        start = row_base + c * _CHUNK
        pltpu.sync_copy(ids_hbm.at[pl.ds(start, _CHUNK)], idx_v)
        for j in range(_GSUB):
            pltpu.async_copy(
                table_hbm.at[idx_v.at[pl.ds(j * _IDXW, _IDXW)]],
                rows_v.at[pl.ds(j * _IDXW, _IDXW)],
                sem_g,
            )
        for j in range(_GSUB):
            pltpu.make_async_copy(
                table_hbm.at[idx_v.at[pl.ds(j * _IDXW, _IDXW)]],
                rows_v.at[pl.ds(j * _IDXW, _IDXW)],
                sem_g,
            ).wait()

        def add_body(r, carry2):
            for j in range(_VPR):
                sl = pl.ds(j * _LANES, _LANES)
                rows_v[r, sl] = rows_v[r, sl] + pe_v[r, sl]
            return carry2

        lax.fori_loop(0, _CHUNK, add_body, 0, unroll=2)

        pltpu.async_copy(rows_v, out_hbm.at[pl.ds(start, _CHUNK)], sem_o).wait()
        return carry

    lax.fori_loop(0, _CHUNKS_PER_W, chunk_body, 0)


_mesh = plsc.VectorSubcoreMesh(core_axis_name="c", subcore_axis_name="s")

_sc_call = pl.kernel(
    _sc_body,
    out_type=jax.ShapeDtypeStruct((_TOTAL_ROWS, _D), jnp.float32),
    mesh=_mesh,
    scratch_types=[
        pltpu.VMEM((_CHUNK, _D), jnp.float32),
        pltpu.VMEM((_CHUNK,), jnp.int32),
        pltpu.VMEM((_CHUNK, _D), jnp.float32),
        pltpu.SemaphoreType.DMA,
        pltpu.SemaphoreType.DMA,
    ],
    compiler_params=pltpu.CompilerParams(use_tc_tiling_on_sc=False),
)


@jax.jit
def kernel(input_ids, token_table):
    ids = input_ids.reshape(_TOTAL_ROWS)
    pe2 = jnp.asarray(_PE2)
    out = _sc_call(ids, pe2, token_table)
    return out.reshape(_BATCH, _SEQ, _D)


# speedup vs baseline: 1.1716x; 1.1716x over previous
import math

import jax
import jax.numpy as jnp
import numpy as np
from jax import lax
from jax.experimental import pallas as pl
from jax.experimental.pallas import tpu as pltpu
from jax.experimental.pallas import tpu_sc as plsc

_VOCAB = 1000000
_D = 64
_MAX_LEN = 512
_BATCH = 4096
_SEQ = 200

_NC = 2
_NS = 16
_NW = _NC * _NS

_TOTAL_ROWS = _BATCH * _SEQ
_ROWS_PER_W = _TOTAL_ROWS // _NW
_CHUNK = 400
_IDXW = 80
_GSUB = _CHUNK // _IDXW
_CHUNKS_PER_W = _ROWS_PER_W // _CHUNK
_LANES = 16
_VPR = _D // _LANES


def _build_pe_np(max_len, d_model):
    pe = np.zeros((max_len, d_model), dtype=np.float32)
    position = np.arange(0, max_len, dtype=np.float32)[:, None]
    div_term = np.exp(
        np.arange(0, d_model, 2, dtype=np.float32) * (-math.log(10000.0) / d_model)
    )
    pe[:, 0::2] = np.sin(position * div_term)
    pe[:, 1::2] = np.cos(position * div_term)
    return pe


_PE2 = np.tile(_build_pe_np(_MAX_LEN, _D)[:_SEQ], (2, 1))


def _sc_body(ids_hbm, pe_hbm, table_hbm, out_hbm,
             idx_v, rows_v, sem_g, sem_o):
    wid = lax.axis_index("s") * _NC + lax.axis_index("c")
    row_base = wid * _ROWS_PER_W

    def chunk_body(c, carry):
        start = row_base + c * _CHUNK
        pltpu.sync_copy(ids_hbm.at[pl.ds(start, _CHUNK)], idx_v)
        pltpu.sync_copy(pe_hbm, rows_v)
        for j in range(_GSUB):
            pltpu.async_copy(
                table_hbm.at[idx_v.at[pl.ds(j * _IDXW, _IDXW)]],
                rows_v.at[pl.ds(j * _IDXW, _IDXW)],
                sem_g,
                add=True,
            )
        for j in range(_GSUB):
            pltpu.make_async_copy(
                table_hbm.at[idx_v.at[pl.ds(j * _IDXW, _IDXW)]],
                rows_v.at[pl.ds(j * _IDXW, _IDXW)],
                sem_g,
            ).wait()

        pltpu.async_copy(rows_v, out_hbm.at[pl.ds(start, _CHUNK)], sem_o).wait()
        return carry

    lax.fori_loop(0, _CHUNKS_PER_W, chunk_body, 0)


_mesh = plsc.VectorSubcoreMesh(core_axis_name="c", subcore_axis_name="s")

_sc_call = pl.kernel(
    _sc_body,
    out_type=jax.ShapeDtypeStruct((_TOTAL_ROWS, _D), jnp.float32),
    mesh=_mesh,
    scratch_types=[
        pltpu.VMEM((_CHUNK,), jnp.int32),
        pltpu.VMEM((_CHUNK, _D), jnp.float32),
        pltpu.SemaphoreType.DMA,
        pltpu.SemaphoreType.DMA,
    ],
    compiler_params=pltpu.CompilerParams(use_tc_tiling_on_sc=False),
)


@jax.jit
def kernel(input_ids, token_table):
    ids = input_ids.reshape(_TOTAL_ROWS)
    pe2 = jnp.asarray(_PE2)
    out = _sc_call(ids, pe2, token_table)
    return out.reshape(_BATCH, _SEQ, _D)


# speedup vs baseline: 1.4238x; 1.2153x over previous
import math

import jax
import jax.numpy as jnp
import numpy as np
from jax import lax
from jax.experimental import pallas as pl
from jax.experimental.pallas import tpu as pltpu
from jax.experimental.pallas import tpu_sc as plsc

_VOCAB = 1000000
_D = 64
_MAX_LEN = 512
_BATCH = 4096
_SEQ = 200

_NC = 2
_NS = 16
_NW = _NC * _NS

_TOTAL_ROWS = _BATCH * _SEQ
_ROWS_PER_W = _TOTAL_ROWS // _NW
_CHUNK = 400
_IDXW = 80
_GSUB = _CHUNK // _IDXW
_NCHUNK = _ROWS_PER_W // _CHUNK


def _build_pe_np(max_len, d_model):
    pe = np.zeros((max_len, d_model), dtype=np.float32)
    position = np.arange(0, max_len, dtype=np.float32)[:, None]
    div_term = np.exp(
        np.arange(0, d_model, 2, dtype=np.float32) * (-math.log(10000.0) / d_model)
    )
    pe[:, 0::2] = np.sin(position * div_term)
    pe[:, 1::2] = np.cos(position * div_term)
    return pe


_PE2 = np.tile(_build_pe_np(_MAX_LEN, _D)[:_SEQ], (2, 1))


def _sc_body(ids_hbm, pe_hbm, table_hbm, out_hbm,
             pe_sh, idx0, idx1, rows0, rows1,
             sem_x0, sem_x1, sem_i0, sem_i1,
             sem_g0, sem_g1, sem_o0, sem_o1):
    wid = lax.axis_index("s") * _NC + lax.axis_index("c")
    row_base = wid * _ROWS_PER_W

    @pl.when(lax.axis_index("s") == 0)
    def _():
        pltpu.sync_copy(pe_hbm, pe_sh)

    plsc.subcore_barrier()

    idx = (idx0, idx1)
    rows = (rows0, rows1)
    sem_x = (sem_x0, sem_x1)
    sem_i = (sem_i0, sem_i1)
    sem_g = (sem_g0, sem_g1)
    sem_o = (sem_o0, sem_o1)

    def issue_stage(c, s):
        start = row_base + c * _CHUNK
        pltpu.async_copy(ids_hbm.at[pl.ds(start, _CHUNK)], idx[s], sem_x[s])
        pltpu.async_copy(pe_sh, rows[s], sem_i[s])

    def wait_issue(c, s):
        pltpu.make_async_copy(ids_hbm.at[pl.ds(0, _CHUNK)], idx[s], sem_x[s]).wait()
        pltpu.make_async_copy(pe_sh, rows[s], sem_i[s]).wait()

    def gather_stage(c, s):
        wait_issue(c, s)
        for j in range(_GSUB):
            pltpu.async_copy(
                table_hbm.at[idx[s].at[pl.ds(j * _IDXW, _IDXW)]],
                rows[s].at[pl.ds(j * _IDXW, _IDXW)],
                sem_g[s],
                add=True,
            )

    def out_stage(c, s):
        start = row_base + c * _CHUNK
        for j in range(_GSUB):
            pltpu.make_async_copy(
                table_hbm.at[idx[s].at[pl.ds(j * _IDXW, _IDXW)]],
                rows[s].at[pl.ds(j * _IDXW, _IDXW)],
                sem_g[s],
            ).wait()
        pltpu.async_copy(rows[s], out_hbm.at[pl.ds(start, _CHUNK)], sem_o[s])

    def wait_out(c, s):
        start = row_base + c * _CHUNK
        pltpu.make_async_copy(rows[s], out_hbm.at[pl.ds(start, _CHUNK)], sem_o[s]).wait()

    issue_stage(0, 0)
    gather_stage(0, 0)
    issue_stage(1, 1)

    def pair_body(i, carry):
        c = 2 * i
        gather_stage(c + 1, 1)
        out_stage(c, 0)
        wait_out(c, 0)
        issue_stage(c + 2, 0)
        gather_stage(c + 2, 0)
        out_stage(c + 1, 1)
        wait_out(c + 1, 1)
        issue_stage(c + 3, 1)
        return carry

    lax.fori_loop(0, (_NCHUNK - 2) // 2, pair_body, 0)

    c_last = _NCHUNK - 2
    gather_stage(c_last + 1, 1)
    out_stage(c_last, 0)
    out_stage(c_last + 1, 1)
    wait_out(c_last, 0)
    wait_out(c_last + 1, 1)


_mesh = plsc.VectorSubcoreMesh(core_axis_name="c", subcore_axis_name="s")

_sc_call = pl.kernel(
    _sc_body,
    out_type=jax.ShapeDtypeStruct((_TOTAL_ROWS, _D), jnp.float32),
    mesh=_mesh,
    scratch_types=[
        pltpu.VMEM_SHARED((_CHUNK, _D), jnp.float32),
        pltpu.VMEM((_CHUNK,), jnp.int32),
        pltpu.VMEM((_CHUNK,), jnp.int32),
        pltpu.VMEM((_CHUNK, _D), jnp.float32),
        pltpu.VMEM((_CHUNK, _D), jnp.float32),
        pltpu.SemaphoreType.DMA,
        pltpu.SemaphoreType.DMA,
        pltpu.SemaphoreType.DMA,
        pltpu.SemaphoreType.DMA,
        pltpu.SemaphoreType.DMA,
        pltpu.SemaphoreType.DMA,
        pltpu.SemaphoreType.DMA,
        pltpu.SemaphoreType.DMA,
    ],
    compiler_params=pltpu.CompilerParams(use_tc_tiling_on_sc=False),
)


@jax.jit
def kernel(input_ids, token_table):
    ids = input_ids.reshape(_TOTAL_ROWS)
    pe2 = jnp.asarray(_PE2)
    out = _sc_call(ids, pe2, token_table)
    return out.reshape(_BATCH, _SEQ, _D)


# speedup vs baseline: 1.5009x; 1.0542x over previous
import math

import jax
import jax.numpy as jnp
import numpy as np
from jax import lax
from jax.experimental import pallas as pl
from jax.experimental.pallas import tpu as pltpu
from jax.experimental.pallas import tpu_sc as plsc

_VOCAB = 1000000
_D = 64
_MAX_LEN = 512
_BATCH = 4096
_SEQ = 200

_NC = 2
_NS = 16
_NW = _NC * _NS

_TOTAL_ROWS = _BATCH * _SEQ
_ROWS_PER_W = _TOTAL_ROWS // _NW
_CHUNK = 400
_IDXW = 80
_GSUB = _CHUNK // _IDXW
_NCHUNK = _ROWS_PER_W // _CHUNK


def _build_pe_np(max_len, d_model):
    pe = np.zeros((max_len, d_model), dtype=np.float32)
    position = np.arange(0, max_len, dtype=np.float32)[:, None]
    div_term = np.exp(
        np.arange(0, d_model, 2, dtype=np.float32) * (-math.log(10000.0) / d_model)
    )
    pe[:, 0::2] = np.sin(position * div_term)
    pe[:, 1::2] = np.cos(position * div_term)
    return pe


_PE2 = np.tile(_build_pe_np(_MAX_LEN, _D)[:_SEQ], (2, 1))


def _sc_body(ids_hbm, pe_hbm, table_hbm, out_hbm,
             pe_sh, idx0, idx1, rows0, rows1,
             sem_x0, sem_x1, sem_i0, sem_i1,
             sem_g0, sem_g1, sem_o0, sem_o1):
    wid = lax.axis_index("s") * _NC + lax.axis_index("c")
    row_base = wid * _ROWS_PER_W

    @pl.when(lax.axis_index("s") == 0)
    def _():
        pltpu.sync_copy(pe_hbm, pe_sh)

    plsc.subcore_barrier()

    idx = (idx0, idx1)
    rows = (rows0, rows1)
    sem_x = (sem_x0, sem_x1)
    sem_i = (sem_i0, sem_i1)
    sem_g = (sem_g0, sem_g1)
    sem_o = (sem_o0, sem_o1)

    def issue_stage(c, s):
        start = row_base + c * _CHUNK
        pltpu.async_copy(ids_hbm.at[pl.ds(start, _CHUNK)], idx[s], sem_x[s])
        pltpu.async_copy(pe_sh, rows[s], sem_i[s])

    def wait_issue(c, s):
        pltpu.make_async_copy(ids_hbm.at[pl.ds(0, _CHUNK)], idx[s], sem_x[s]).wait()
        pltpu.make_async_copy(pe_sh, rows[s], sem_i[s]).wait()

    def gather_stage(c, s):
        wait_issue(c, s)
        for j in range(_GSUB):
            pltpu.async_copy(
                table_hbm.at[idx[s].at[pl.ds(j * _IDXW, _IDXW)]],
                rows[s].at[pl.ds(j * _IDXW, _IDXW)],
                sem_g[s],
                add=True,
            )

    def out_stage(c, s):
        start = row_base + c * _CHUNK
        for j in range(_GSUB):
            pltpu.make_async_copy(
                table_hbm.at[idx[s].at[pl.ds(j * _IDXW, _IDXW)]],
                rows[s].at[pl.ds(j * _IDXW, _IDXW)],
                sem_g[s],
            ).wait()
        pltpu.async_copy(rows[s], out_hbm.at[pl.ds(start, _CHUNK)], sem_o[s])

    def wait_out(c, s):
        start = row_base + c * _CHUNK
        pltpu.make_async_copy(rows[s], out_hbm.at[pl.ds(start, _CHUNK)], sem_o[s]).wait()

    issue_stage(0, 0)
    gather_stage(0, 0)
    issue_stage(1, 1)

    def pair_body(i, carry):
        c = 2 * i
        gather_stage(c + 1, 1)
        out_stage(c, 0)
        wait_out(c, 0)
        issue_stage(c + 2, 0)
        gather_stage(c + 2, 0)
        out_stage(c + 1, 1)
        wait_out(c + 1, 1)
        issue_stage(c + 3, 1)
        return carry

    lax.fori_loop(0, (_NCHUNK - 2) // 2, pair_body, 0)

    c_last = _NCHUNK - 2
    gather_stage(c_last + 1, 1)
    out_stage(c_last, 0)
    out_stage(c_last + 1, 1)
    wait_out(c_last, 0)
    wait_out(c_last + 1, 1)


_mesh = plsc.VectorSubcoreMesh(core_axis_name="c", subcore_axis_name="s")

_sc_call = pl.kernel(
    _sc_body,
    out_type=jax.ShapeDtypeStruct((_TOTAL_ROWS, _D), jnp.float32),

    mesh=_mesh,
    scratch_types=[
        pltpu.VMEM_SHARED((_CHUNK, _D), jnp.float32),
        pltpu.VMEM((_CHUNK,), jnp.int32),
        pltpu.VMEM((_CHUNK,), jnp.int32),
        pltpu.VMEM((_CHUNK, _D), jnp.float32),
        pltpu.VMEM((_CHUNK, _D), jnp.float32),
        pltpu.SemaphoreType.DMA,
        pltpu.SemaphoreType.DMA,
        pltpu.SemaphoreType.DMA,
        pltpu.SemaphoreType.DMA,
        pltpu.SemaphoreType.DMA,
        pltpu.SemaphoreType.DMA,
        pltpu.SemaphoreType.DMA,
        pltpu.SemaphoreType.DMA,
    ],
    compiler_params=pltpu.CompilerParams(use_tc_tiling_on_sc=False),
)


@jax.jit
def kernel(input_ids, token_table):
    ids2 = input_ids.reshape(_TOTAL_ROWS) * 2
    tblp = jnp.pad(token_table, ((0, 0), (0, _D))).reshape(2 * _VOCAB, _D)
    pe2 = jnp.asarray(_PE2)
    out = _sc_call(ids2, pe2, tblp)
    return out.reshape(_BATCH, _SEQ, _D)


# speedup vs baseline: 1.7484x; 1.1649x over previous
import math

import jax
import jax.numpy as jnp
import numpy as np
from jax import lax
from jax.experimental import pallas as pl
from jax.experimental.pallas import tpu as pltpu
from jax.experimental.pallas import tpu_sc as plsc

_VOCAB = 1000000
_D = 64
_DP = 128
_MAX_LEN = 512
_BATCH = 4096
_SEQ = 200

_NC = 2
_NS = 16
_NW = _NC * _NS

_TOTAL_ROWS = _BATCH * _SEQ
_ROWS_PER_W = _TOTAL_ROWS // _NW
_CHUNK = 400
_IDXW = 80
_GSUB = _CHUNK // _IDXW
_NCHUNK = _ROWS_PER_W // _CHUNK


def _build_pe_np(max_len, d_model):
    pe = np.zeros((max_len, d_model), dtype=np.float32)
    position = np.arange(0, max_len, dtype=np.float32)[:, None]
    div_term = np.exp(
        np.arange(0, d_model, 2, dtype=np.float32) * (-math.log(10000.0) / d_model)
    )
    pe[:, 0::2] = np.sin(position * div_term)
    pe[:, 1::2] = np.cos(position * div_term)
    return pe


_PE2P = np.zeros((_CHUNK, _DP), dtype=np.float32)
_PE2P[:, :_D] = np.tile(_build_pe_np(_MAX_LEN, _D)[:_SEQ], (2, 1))


def _sc_body(ids_hbm, pe_hbm, table_hbm, out_hbm,
             pe_sh, idx0, idx1, rows0, rows1,
             sem_x0, sem_x1, sem_i0, sem_i1,
             sem_g0, sem_g1, sem_o0, sem_o1):
    wid = lax.axis_index("s") * _NC + lax.axis_index("c")
    row_base = wid * _ROWS_PER_W

    @pl.when(lax.axis_index("s") == 0)
    def _():
        pltpu.sync_copy(pe_hbm, pe_sh)

    plsc.subcore_barrier()

    idx = (idx0, idx1)
    rows = (rows0, rows1)
    sem_x = (sem_x0, sem_x1)
    sem_i = (sem_i0, sem_i1)
    sem_g = (sem_g0, sem_g1)
    sem_o = (sem_o0, sem_o1)

    def issue_stage(c, s):
        start = row_base + c * _CHUNK
        pltpu.async_copy(ids_hbm.at[pl.ds(start, _CHUNK)], idx[s], sem_x[s])
        pltpu.async_copy(pe_sh, rows[s], sem_i[s])

    def wait_issue(c, s):
        pltpu.make_async_copy(ids_hbm.at[pl.ds(0, _CHUNK)], idx[s], sem_x[s]).wait()
        pltpu.make_async_copy(pe_sh, rows[s], sem_i[s]).wait()

    def gather_stage(c, s):
        wait_issue(c, s)
        for j in range(_GSUB):
            pltpu.async_copy(
                table_hbm.at[idx[s].at[pl.ds(j * _IDXW, _IDXW)]],
                rows[s].at[pl.ds(j * _IDXW, _IDXW)],
                sem_g[s],
                add=True,
            )

    def out_stage(c, s):
        start = row_base + c * _CHUNK
        for j in range(_GSUB):
            pltpu.make_async_copy(
                table_hbm.at[idx[s].at[pl.ds(j * _IDXW, _IDXW)]],
                rows[s].at[pl.ds(j * _IDXW, _IDXW)],
                sem_g[s],
            ).wait()
        pltpu.async_copy(rows[s], out_hbm.at[pl.ds(start, _CHUNK)], sem_o[s])

    def wait_out(c, s):
        start = row_base + c * _CHUNK
        pltpu.make_async_copy(
            rows[s], out_hbm.at[pl.ds(start, _CHUNK)], sem_o[s]
        ).wait()

    issue_stage(0, 0)
    gather_stage(0, 0)
    issue_stage(1, 1)

    def pair_body(i, carry):
        c = 2 * i
        gather_stage(c + 1, 1)
        out_stage(c, 0)
        wait_out(c, 0)
        issue_stage(c + 2, 0)
        gather_stage(c + 2, 0)
        out_stage(c + 1, 1)
        wait_out(c + 1, 1)
        issue_stage(c + 3, 1)
        return carry

    lax.fori_loop(0, (_NCHUNK - 2) // 2, pair_body, 0)

    c_last = _NCHUNK - 2
    gather_stage(c_last + 1, 1)
    out_stage(c_last, 0)
    out_stage(c_last + 1, 1)
    wait_out(c_last, 0)
    wait_out(c_last + 1, 1)


_mesh = plsc.VectorSubcoreMesh(core_axis_name="c", subcore_axis_name="s")

_sc_call = pl.kernel(
    _sc_body,
    out_type=jax.ShapeDtypeStruct((_TOTAL_ROWS, _DP), jnp.float32),
    mesh=_mesh,
    scratch_types=[
        pltpu.VMEM_SHARED((_CHUNK, _DP), jnp.float32),
        pltpu.VMEM((_CHUNK,), jnp.int32),
        pltpu.VMEM((_CHUNK,), jnp.int32),
        pltpu.VMEM((_CHUNK, _DP), jnp.float32),
        pltpu.VMEM((_CHUNK, _DP), jnp.float32),
        pltpu.SemaphoreType.DMA,
        pltpu.SemaphoreType.DMA,
        pltpu.SemaphoreType.DMA,
        pltpu.SemaphoreType.DMA,
        pltpu.SemaphoreType.DMA,
        pltpu.SemaphoreType.DMA,
        pltpu.SemaphoreType.DMA,
        pltpu.SemaphoreType.DMA,
    ],
    compiler_params=pltpu.CompilerParams(use_tc_tiling_on_sc=True),
)


@jax.jit
def kernel(input_ids, token_table):
    ids = input_ids.reshape(_TOTAL_ROWS)
    tblp = jnp.pad(token_table, ((0, 0), (0, _DP - _D)))
    pe2p = jnp.asarray(_PE2P)
    out = _sc_call(ids, pe2p, tblp)
    return out[:, :_D].reshape(_BATCH, _SEQ, _D)
